# baseline (speedup 1.0000x reference)
import functools

import jax
import jax.numpy as jnp
from jax import lax
from jax.experimental import pallas as pl
from jax.experimental.pallas import tpu as pltpu
from jax.experimental.pallas import tpu_sc as plsc

N = 10000
E = 320000
D = 128

NW = 32
NP = 10240
EP = 327680
EPT = EP // NW
CH = 128
NCH = EPT // CH
CHUNKS = EP // CH
RPT = NP // 16
NCH0 = 160
NCH1 = 0

_mesh = plsc.VectorSubcoreMesh(core_axis_name="c", subcore_axis_name="s")


@functools.partial(
    pl.kernel,
    out_type=jax.ShapeDtypeStruct((2, 2, NP), jnp.float32),
    mesh=_mesh,
    scratch_types=[
        pltpu.VMEM((NCH, CH), jnp.int32),
        pltpu.VMEM((NCH, CH), jnp.int32),
        pltpu.VMEM((CH,), jnp.float32),
        pltpu.VMEM((CH,), jnp.float32),
        pltpu.VMEM_SHARED((NP,), jnp.float32),
        pltpu.VMEM_SHARED((NP,), jnp.float32),
        pltpu.SemaphoreType.DMA,
    ],
)
def _sc_degrees(src_hbm, dst_hbm, ones_hbm, out_hbm,
                sidx2, didx2, onev, zv, cnt_o, cnt_i, sd):
    c = lax.axis_index("c")
    s = lax.axis_index("s")
    wid = c * 16 + s
    for k in range(CH // 16):
        zv[pl.ds(k * 16, 16)] = jnp.zeros((16,), jnp.float32)
    for k in range(RPT // CH):
        off = s * RPT + k * CH
        pltpu.sync_copy(zv, cnt_o.at[pl.ds(off, CH)])
        pltpu.sync_copy(zv, cnt_i.at[pl.ds(off, CH)])
    pltpu.sync_copy(ones_hbm, onev)
    pltpu.sync_copy(src_hbm.at[pl.ds(wid * NCH, NCH)], sidx2)
    pltpu.sync_copy(dst_hbm.at[pl.ds(wid * NCH, NCH)], didx2)
    plsc.subcore_barrier()

    GRP = 8

    def body(i, _):
        for b in range(GRP):
            jj = i * GRP + b
            pltpu.async_copy(onev, cnt_o.at[sidx2.at[jj]], sd, add=True)
            pltpu.async_copy(onev, cnt_i.at[didx2.at[jj]], sd, add=True)
        for b in range(GRP):
            jj = i * GRP + b
            pltpu.make_async_copy(onev, cnt_o.at[sidx2.at[jj]], sd).wait()
            pltpu.make_async_copy(onev, cnt_i.at[didx2.at[jj]], sd).wait()
        return 0

    lax.fori_loop(0, NCH // GRP, body, 0)
    plsc.subcore_barrier()

    @pl.when(s == 0)
    def _():
        pltpu.sync_copy(cnt_o, out_hbm.at[c, 0])
        pltpu.sync_copy(cnt_i, out_hbm.at[c, 1])


NBUF = 2
NIB = 4


@functools.partial(
    pl.kernel,
    out_type=jax.ShapeDtypeStruct((2, NP, D), jnp.float32),
    mesh=_mesh,
    scratch_types=[
        pltpu.VMEM((NIB, CH), jnp.int32),
        pltpu.VMEM((NIB, CH), jnp.int32),
        pltpu.VMEM((NBUF, CH, D), jnp.float32),
        pltpu.VMEM_SHARED((NP, D), jnp.float32),
        [pltpu.SemaphoreType.DMA] * NBUF,
        [pltpu.SemaphoreType.DMA] * NBUF,
        [pltpu.SemaphoreType.DMA] * NIB,
    ],
)
def _sc_spmm(x_hbm, src_hbm, dst_hbm, out_hbm,
             sidxr, didxr, rows, agg, sg, ss, si):
    c = lax.axis_index("c")
    s = lax.axis_index("s")
    zbuf = rows.at[0]

    def zrow(r, _):
        for k in range(D // 16):
            zbuf[r, pl.ds(k * 16, 16)] = jnp.zeros((16,), jnp.float32)
        return 0

    lax.fori_loop(0, CH, zrow, 0)
    for k in range(RPT // CH):
        pltpu.sync_copy(zbuf, agg.at[pl.ds(s * RPT + k * CH, CH)])
    plsc.subcore_barrier()

    def idx_start(jj, ib):
        pltpu.async_copy(src_hbm.at[jj], sidxr.at[ib], si[ib])
        pltpu.async_copy(dst_hbm.at[jj], didxr.at[ib], si[ib])

    def idx_wait(jj, ib):
        pltpu.make_async_copy(src_hbm.at[jj], sidxr.at[ib], si[ib]).wait()
        pltpu.make_async_copy(dst_hbm.at[jj], didxr.at[ib], si[ib]).wait()

    def gather_start(ib, b):
        pltpu.async_copy(x_hbm.at[sidxr.at[ib]], rows.at[b], sg[b])

    def gather_wait(ib, b):
        pltpu.make_async_copy(x_hbm.at[sidxr.at[ib]], rows.at[b], sg[b]).wait()

    def scatter_start(ib, b):
        pltpu.async_copy(rows.at[b], agg.at[didxr.at[ib]], ss[b], add=True)

    def scatter_wait(ib, b):
        pltpu.make_async_copy(rows.at[b], agg.at[didxr.at[ib]], ss[b]).wait()

    def pipe(nch, base):
        idx_start(base + 0, 0)
        idx_start(base + 1, 1)
        idx_wait(base + 0, 0)
        gather_start(0, 0)

        def body(i, _):
            for b in range(NIB):
                jj = i * NIB + b
                f = jj + 1
                fib = (b + 1) % NIB
                f2 = (b + 1) % NBUF

                @pl.when(f < nch)
                def _():
                    idx_wait(base + f, fib)

                    @pl.when(jj >= 1)
                    def _():
                        scatter_wait((b - 1) % NIB, f2)

                    gather_start(fib, f2)

                gather_wait(b, b % NBUF)

                @pl.when(jj + 2 < nch)
                def _():
                    idx_start(base + jj + 2, (b + 2) % NIB)

                scatter_start(b, b % NBUF)
            return 0

        lax.fori_loop(0, nch // NIB, body, 0)
        scatter_wait((nch - 2) % NIB, (nch - 2) % NBUF)
        scatter_wait((nch - 1) % NIB, (nch - 1) % NBUF)

    if NCH0:
        @pl.when(c == 0)
        def _():
            pipe(NCH0, s * NCH0)

    if NCH1:
        @pl.when(c == 1)
        def _():
            pipe(NCH1, 16 * NCH0 + s * NCH1)

    plsc.subcore_barrier()
    pltpu.sync_copy(agg.at[pl.ds(s * RPT, RPT)], out_hbm.at[c, pl.ds(s * RPT, RPT)])


def _tc1_body(h_ref, w_ref, co_ref, o_ref):
    so = lax.rsqrt(jnp.maximum(co_ref[...], 1.0))
    o_ref[...] = jnp.dot(h_ref[...] * so, w_ref[...],
                         preferred_element_type=jnp.float32)


def _tc2_body(aggp_ref, ci_ref, co_ref, b1_ref, w2_ref, o_ref):
    agg = aggp_ref[0] + aggp_ref[1]
    si = lax.rsqrt(jnp.maximum(ci_ref[...], 1.0))
    so = lax.rsqrt(jnp.maximum(co_ref[...], 1.0))
    h1 = jnp.maximum(agg * si + b1_ref[...], 0.0)
    o_ref[...] = jnp.dot(h1 * so, w2_ref[...],
                         preferred_element_type=jnp.float32)


def _tc3_body(aggp_ref, ci_ref, b2_ref, o_ref):
    agg = aggp_ref[0] + aggp_ref[1]
    si = lax.rsqrt(jnp.maximum(ci_ref[...], 1.0))
    o_ref[...] = agg * si + b2_ref[...]


_tc1 = pl.pallas_call(
    _tc1_body, out_shape=jax.ShapeDtypeStruct((NP, D), jnp.float32))
_tc2 = pl.pallas_call(
    _tc2_body, out_shape=jax.ShapeDtypeStruct((NP, D), jnp.float32))
_tc3 = pl.pallas_call(
    _tc3_body, out_shape=jax.ShapeDtypeStruct((NP, D), jnp.float32))


def kernel(h, edge_index, W1, b1, W2, b2):
    src = edge_index[0].astype(jnp.int32)
    dst = edge_index[1].astype(jnp.int32)
    pad = jnp.full((EP - E,), N, jnp.int32)
    srcp = jnp.concatenate([src, pad]).reshape(CHUNKS, CH)
    dstp = jnp.concatenate([dst, pad]).reshape(CHUNKS, CH)
    hp = jnp.pad(h, ((0, NP - N), (0, 0)))
    ones1 = jnp.ones((CH,), jnp.float32)

    cnt = _sc_degrees(srcp, dstp, ones1)
    co = (cnt[0, 0] + cnt[1, 0]).reshape(NP, 1)
    ci = (cnt[0, 1] + cnt[1, 1]).reshape(NP, 1)

    x1 = _tc1(hp, W1, co)
    agg1 = _sc_spmm(x1, srcp, dstp)
    x2 = _tc2(agg1, ci, co, b1.reshape(1, D), W2)
    agg2 = _sc_spmm(x2, srcp, dstp)
    out = _tc3(agg2, ci, b2.reshape(1, D))
    return out[:N]


# speedup vs baseline: 1.0961x; 1.0961x over previous
import functools

import jax
import jax.numpy as jnp
from jax import lax
from jax.experimental import pallas as pl
from jax.experimental.pallas import tpu as pltpu
from jax.experimental.pallas import tpu_sc as plsc

N = 10000
E = 320000
D = 128

NW = 32
NP = 10240
EP = 327680
EPT = EP // NW
CH = 128
NCH = EPT // CH
CHUNKS = EP // CH
RPT = NP // 16
NCH0 = 80
NCH1 = 80

_mesh = plsc.VectorSubcoreMesh(core_axis_name="c", subcore_axis_name="s")


@functools.partial(
    pl.kernel,
    out_type=jax.ShapeDtypeStruct((2, 2, NP), jnp.float32),
    mesh=_mesh,
    scratch_types=[
        pltpu.VMEM((NCH, CH), jnp.int32),
        pltpu.VMEM((NCH, CH), jnp.int32),
        pltpu.VMEM((CH,), jnp.float32),
        pltpu.VMEM((CH,), jnp.float32),
        pltpu.VMEM_SHARED((NP,), jnp.float32),
        pltpu.VMEM_SHARED((NP,), jnp.float32),
        pltpu.SemaphoreType.DMA,
    ],
)
def _sc_degrees(src_hbm, dst_hbm, ones_hbm, out_hbm,
                sidx2, didx2, onev, zv, cnt_o, cnt_i, sd):
    c = lax.axis_index("c")
    s = lax.axis_index("s")
    wid = c * 16 + s
    for k in range(CH // 16):
        zv[pl.ds(k * 16, 16)] = jnp.zeros((16,), jnp.float32)
    for k in range(RPT // CH):
        off = s * RPT + k * CH
        pltpu.sync_copy(zv, cnt_o.at[pl.ds(off, CH)])
        pltpu.sync_copy(zv, cnt_i.at[pl.ds(off, CH)])
    pltpu.sync_copy(ones_hbm, onev)
    pltpu.sync_copy(src_hbm.at[pl.ds(wid * NCH, NCH)], sidx2)
    pltpu.sync_copy(dst_hbm.at[pl.ds(wid * NCH, NCH)], didx2)
    plsc.subcore_barrier()

    GRP = 8

    def body(i, _):
        for b in range(GRP):
            jj = i * GRP + b
            pltpu.async_copy(onev, cnt_o.at[sidx2.at[jj]], sd, add=True)
            pltpu.async_copy(onev, cnt_i.at[didx2.at[jj]], sd, add=True)
        for b in range(GRP):
            jj = i * GRP + b
            pltpu.make_async_copy(onev, cnt_o.at[sidx2.at[jj]], sd).wait()
            pltpu.make_async_copy(onev, cnt_i.at[didx2.at[jj]], sd).wait()
        return 0

    lax.fori_loop(0, NCH // GRP, body, 0)
    plsc.subcore_barrier()

    @pl.when(s == 0)
    def _():
        pltpu.sync_copy(cnt_o, out_hbm.at[c, 0])
        pltpu.sync_copy(cnt_i, out_hbm.at[c, 1])


NBUF = 2
NIB = 4


@functools.partial(
    pl.kernel,
    out_type=jax.ShapeDtypeStruct((2, NP, D), jnp.float32),
    mesh=_mesh,
    scratch_types=[
        pltpu.VMEM((NIB, CH), jnp.int32),
        pltpu.VMEM((NIB, CH), jnp.int32),
        pltpu.VMEM((NBUF, CH, D), jnp.float32),
        pltpu.VMEM_SHARED((NP, D), jnp.float32),
        [pltpu.SemaphoreType.DMA] * NBUF,
        [pltpu.SemaphoreType.DMA] * NBUF,
        [pltpu.SemaphoreType.DMA] * NIB,
    ],
)
def _sc_spmm(x_hbm, src_hbm, dst_hbm, out_hbm,
             sidxr, didxr, rows, agg, sg, ss, si):
    c = lax.axis_index("c")
    s = lax.axis_index("s")
    zbuf = rows.at[0]

    def zrow(r, _):
        for k in range(D // 16):
            zbuf[r, pl.ds(k * 16, 16)] = jnp.zeros((16,), jnp.float32)
        return 0

    lax.fori_loop(0, CH, zrow, 0)
    for k in range(RPT // CH):
        pltpu.sync_copy(zbuf, agg.at[pl.ds(s * RPT + k * CH, CH)])
    plsc.subcore_barrier()

    def idx_start(jj, ib):
        pltpu.async_copy(src_hbm.at[jj], sidxr.at[ib], si[ib])
        pltpu.async_copy(dst_hbm.at[jj], didxr.at[ib], si[ib])

    def idx_wait(jj, ib):
        pltpu.make_async_copy(src_hbm.at[jj], sidxr.at[ib], si[ib]).wait()
        pltpu.make_async_copy(dst_hbm.at[jj], didxr.at[ib], si[ib]).wait()

    def gather_start(ib, b):
        pltpu.async_copy(x_hbm.at[sidxr.at[ib]], rows.at[b], sg[b])

    def gather_wait(ib, b):
        pltpu.make_async_copy(x_hbm.at[sidxr.at[ib]], rows.at[b], sg[b]).wait()

    def scatter_start(ib, b):
        pltpu.async_copy(rows.at[b], agg.at[didxr.at[ib]], ss[b], add=True)

    def scatter_wait(ib, b):
        pltpu.make_async_copy(rows.at[b], agg.at[didxr.at[ib]], ss[b]).wait()

    def pipe(nch, base):
        idx_start(base + 0, 0)
        idx_start(base + 1, 1)
        idx_wait(base + 0, 0)
        gather_start(0, 0)

        def body(i, _):
            for b in range(NIB):
                jj = i * NIB + b
                f = jj + 1
                fib = (b + 1) % NIB
                f2 = (b + 1) % NBUF

                @pl.when(f < nch)
                def _():
                    idx_wait(base + f, fib)

                    @pl.when(jj >= 1)
                    def _():
                        scatter_wait((b - 1) % NIB, f2)

                    gather_start(fib, f2)

                gather_wait(b, b % NBUF)

                @pl.when(jj + 2 < nch)
                def _():
                    idx_start(base + jj + 2, (b + 2) % NIB)

                scatter_start(b, b % NBUF)
            return 0

        lax.fori_loop(0, nch // NIB, body, 0)
        scatter_wait((nch - 2) % NIB, (nch - 2) % NBUF)
        scatter_wait((nch - 1) % NIB, (nch - 1) % NBUF)

    @pl.when(c == 0)
    def _():
        pipe(NCH0, s * NCH0)

    @pl.when(c == 1)
    def _():
        pipe(NCH1, 16 * NCH0 + s * NCH1)

    plsc.subcore_barrier()
    pltpu.sync_copy(agg.at[pl.ds(s * RPT, RPT)], out_hbm.at[c, pl.ds(s * RPT, RPT)])


def _tc1_body(h_ref, w_ref, co_ref, o_ref):
    so = lax.rsqrt(jnp.maximum(co_ref[...], 1.0))
    o_ref[...] = jnp.dot(h_ref[...] * so, w_ref[...],
                         preferred_element_type=jnp.float32)


def _tc2_body(aggp_ref, ci_ref, co_ref, b1_ref, w2_ref, o_ref):
    agg = aggp_ref[0] + aggp_ref[1]
    si = lax.rsqrt(jnp.maximum(ci_ref[...], 1.0))
    so = lax.rsqrt(jnp.maximum(co_ref[...], 1.0))
    h1 = jnp.maximum(agg * si + b1_ref[...], 0.0)
    o_ref[...] = jnp.dot(h1 * so, w2_ref[...],
                         preferred_element_type=jnp.float32)


def _tc3_body(aggp_ref, ci_ref, b2_ref, o_ref):
    agg = aggp_ref[0] + aggp_ref[1]
    si = lax.rsqrt(jnp.maximum(ci_ref[...], 1.0))
    o_ref[...] = agg * si + b2_ref[...]


_tc1 = pl.pallas_call(
    _tc1_body, out_shape=jax.ShapeDtypeStruct((NP, D), jnp.float32))
_tc2 = pl.pallas_call(
    _tc2_body, out_shape=jax.ShapeDtypeStruct((NP, D), jnp.float32))
_tc3 = pl.pallas_call(
    _tc3_body, out_shape=jax.ShapeDtypeStruct((NP, D), jnp.float32))


def kernel(h, edge_index, W1, b1, W2, b2):
    src = edge_index[0].astype(jnp.int32)
    dst = edge_index[1].astype(jnp.int32)
    pad = jnp.full((EP - E,), N, jnp.int32)
    srcp = jnp.concatenate([src, pad]).reshape(CHUNKS, CH)
    dstp = jnp.concatenate([dst, pad]).reshape(CHUNKS, CH)
    hp = jnp.pad(h, ((0, NP - N), (0, 0)))
    ones1 = jnp.ones((CH,), jnp.float32)

    cnt = _sc_degrees(srcp, dstp, ones1)
    co = (cnt[0, 0] + cnt[1, 0]).reshape(NP, 1)
    ci = (cnt[0, 1] + cnt[1, 1]).reshape(NP, 1)

    x1 = _tc1(hp, W1, co)
    agg1 = _sc_spmm(x1, srcp, dstp)
    x2 = _tc2(agg1, ci, co, b1.reshape(1, D), W2)
    agg2 = _sc_spmm(x2, srcp, dstp)
    out = _tc3(agg2, ci, b2.reshape(1, D))
    return out[:N]
